# baseline (speedup 1.0000x reference)
import jax
import jax.numpy as jnp
from jax.experimental import pallas as pl
from jax.experimental.pallas import tpu as pltpu

_B, _N, _D, _H, _C = 16, 1024, 128, 128, 10
_NSLAB = 4
_SLAB = _N // _NSLAB
_NBUF = 8
_CP = 128


def _slab_copy(adj_ref, abuf_ref, sem, g, buf):
    return pltpu.make_async_copy(
        adj_ref.at[g], abuf_ref.at[buf], sem.at[buf])


def _fused_body(adj_ref, x_ref, wr_ref, wn_ref, b_ref, wc_ref, bc_ref,
                out_ref, abuf_ref, sem):
    bidx = pl.program_id(0)
    xh = x_ref[0]

    @pl.when(bidx == 0)
    def _():
        for g0 in range(2 * _NSLAB):
            _slab_copy(adj_ref, abuf_ref, sem, g0, g0).start()

    base = bidx * _NSLAB
    parts = []
    for s in range(_NSLAB):
        buf = (base + s) % _NBUF
        _slab_copy(adj_ref, abuf_ref, sem, base + s, buf).wait()
        A = (abuf_ref[buf] > 0.5).astype(jnp.bfloat16)
        parts.append(jax.lax.dot_general(
            A, xh[s * _SLAB:(s + 1) * _SLAB],
            dimension_numbers=(((0,), (0,)), ((), ())),
            preferred_element_type=jnp.float32))

        @pl.when(bidx < _B - 2)
        def _():
            _slab_copy(adj_ref, abuf_ref, sem,
                       base + 2 * _NSLAB + s, buf).start()

    agg = (parts[0] + parts[1]) + (parts[2] + parts[3])
    h = jax.lax.dot_general(
        xh, wr_ref[...],
        dimension_numbers=(((1,), (0,)), ((), ())),
        preferred_element_type=jnp.float32)
    h = h + jax.lax.dot_general(
        agg.astype(jnp.bfloat16), wn_ref[...],
        dimension_numbers=(((1,), (0,)), ((), ())),
        preferred_element_type=jnp.float32)
    h = jnp.maximum(h + b_ref[...], 0.0)
    pooled = jnp.sum(h, axis=0, keepdims=True) * (1.0 / _N)
    out_ref[0] = jnp.dot(pooled, wc_ref[...],
                         preferred_element_type=jnp.float32) + bc_ref[...]


def kernel(x, adj, W_root, W_nbr, b, W_cls, b_cls):
    adj_rs = adj.reshape(_B * _NSLAB, _SLAB, _N)
    xh = x.astype(jnp.bfloat16)
    wrh = W_root.astype(jnp.bfloat16)
    wnh = W_nbr.astype(jnp.bfloat16)
    b2 = b.reshape(1, _H)
    wc = jnp.zeros((_H, _CP), jnp.float32).at[:, :_C].set(W_cls)
    bc = jnp.zeros((1, _CP), jnp.float32).at[0, :_C].set(b_cls)

    out = pl.pallas_call(
        _fused_body,
        grid=(_B,),
        in_specs=[
            pl.BlockSpec(memory_space=pltpu.MemorySpace.HBM),
            pl.BlockSpec((1, _N, _D), lambda i: (i, 0, 0)),
            pl.BlockSpec((_D, _H), lambda i: (0, 0)),
            pl.BlockSpec((_D, _H), lambda i: (0, 0)),
            pl.BlockSpec((1, _H), lambda i: (0, 0)),
            pl.BlockSpec((_H, _CP), lambda i: (0, 0)),
            pl.BlockSpec((1, _CP), lambda i: (0, 0)),
        ],
        out_specs=pl.BlockSpec((1, 1, _CP), lambda i: (i, 0, 0)),
        out_shape=jax.ShapeDtypeStruct((_B, 1, _CP), jnp.float32),
        scratch_shapes=[
            pltpu.VMEM((_NBUF, _SLAB, _N), jnp.float32),
            pltpu.SemaphoreType.DMA((_NBUF,)),
        ],
        compiler_params=pltpu.CompilerParams(
            dimension_semantics=("arbitrary",)),
    )(adj_rs, xh, wrh, wnh, b2, wc, bc)
    return out[:, 0, :_C]
